# Initial kernel scaffold; baseline (speedup 1.0000x reference)
import jax
import jax.numpy as jnp
from jax.experimental import pallas as pl


def kernel(input_ids, position_ids, token_type_ids, word_embeddings, position_table, token_type_table, gamma, beta):
    raise NotImplementedError("write your pallas kernel here")


# speedup vs baseline: 3.9086x; 3.9086x over previous
import functools

import jax
import jax.numpy as jnp
from jax import lax
from jax.experimental import pallas as pl
from jax.experimental.pallas import tpu as pltpu
from jax.experimental.pallas import tpu_sc as plsc

VOCAB = 100000
HIDDEN = 128
MAX_POS = 512
EPS = 1e-12
B, L = 1024, 200
N = B * L
NC, NS = 2, 16
NW = NC * NS
PER_W = N // NW
C = 128
G = PER_W // C
NV = HIDDEN // 16
POS_ROWS = 256


def _body(iw_hbm, ip_hbm, it_hbm, w_hbm, p_hbm, t_hbm, g_hbm, b_hbm,
          out_hbm, comb_v, idxw_v, idxp_v, idxt_v, ttv, gamv, betv,
          rows_v, out_v, sem):
    wid = lax.axis_index("s") * NC + lax.axis_index("c")
    base = wid * PER_W

    pw = POS_ROWS * HIDDEN
    pltpu.sync_copy(p_hbm.at[pl.ds(0, pw)], comb_v.at[pl.ds(0, pw)])
    pltpu.sync_copy(p_hbm.at[pl.ds(0, pw)], comb_v.at[pl.ds(pw, pw)])
    pltpu.sync_copy(t_hbm, ttv)
    pltpu.sync_copy(g_hbm, gamv)
    pltpu.sync_copy(b_hbm, betv)

    tt0 = [ttv[0, pl.ds(16 * i, 16)] for i in range(NV)]
    tt1 = [ttv[1, pl.ds(16 * i, 16)] for i in range(NV)]

    @pl.loop(0, POS_ROWS)
    def _comb(r):
        for i in range(NV):
            sl0 = pl.ds(r * HIDDEN + 16 * i, 16)
            sl1 = pl.ds(pw + r * HIDDEN + 16 * i, 16)
            comb_v[sl0] = comb_v[sl0] + tt0[i]
            comb_v[sl1] = comb_v[sl1] + tt1[i]

    gam = [gamv[pl.ds(16 * i, 16)] for i in range(NV)]
    bet = [betv[pl.ds(16 * i, 16)] for i in range(NV)]
    lane = lax.iota(jnp.int32, 16)
    inv_h = jnp.float32(1.0 / HIDDEN)

    @pl.loop(0, G)
    def _chunk(gi):
        co = base + gi * C
        pltpu.sync_copy(iw_hbm.at[pl.ds(co, C)], idxw_v)
        pltpu.sync_copy(ip_hbm.at[pl.ds(co, C)], idxp_v)
        pltpu.sync_copy(it_hbm.at[pl.ds(co, C)], idxt_v)
        pltpu.async_copy(w_hbm.at[idxw_v], rows_v, sem).wait()

        @pl.loop(0, C // 16)
        def _tokgrp(tg):
            pid_vec = idxp_v[pl.ds(16 * tg, 16)]
            tid_vec = idxt_v[pl.ds(16 * tg, 16)]
            ci_vec = (pid_vec + tid_vec * POS_ROWS) * HIDDEN
            for j in range(16):
                t = 16 * tg + j
                ci = ci_vec[j] + lane
                xs = []
                s = None
                q = None
                for i in range(NV):
                    w = rows_v[t, pl.ds(16 * i, 16)]
                    cvec = plsc.load_gather(comb_v, [ci + 16 * i])
                    x = w + cvec
                    xs.append(x)
                    s = x if s is None else s + x
                    q = x * x if q is None else q + x * x
                mean = jnp.sum(s) * inv_h
                var = jnp.maximum(jnp.sum(q) * inv_h - mean * mean, 0.0)
                v = var + jnp.float32(EPS)
                bits = lax.bitcast_convert_type(v, jnp.int32)
                y = lax.bitcast_convert_type(
                    jnp.int32(0x5F3759DF) - (bits >> 1), jnp.float32)
                half_v = 0.5 * v
                for _ in range(3):
                    y = y * (1.5 - half_v * y * y)
                ms = mean * y
                for i in range(NV):
                    out_v[t, pl.ds(16 * i, 16)] = \
                        (xs[i] * y - ms) * gam[i] + bet[i]

        pltpu.sync_copy(out_v, out_hbm.at[pl.ds(co, C)])


@jax.jit
def _run(iw, ip, it, w, p, t, g, b):
    mesh = plsc.VectorSubcoreMesh(core_axis_name="c", subcore_axis_name="s",
                                  num_cores=NC, num_subcores=NS)
    f = pl.kernel(
        _body,
        out_type=jax.ShapeDtypeStruct((N, HIDDEN), jnp.float32),
        mesh=mesh,
        compiler_params=pltpu.CompilerParams(needs_layout_passes=False),
        scratch_types=[
            pltpu.VMEM((2 * POS_ROWS * HIDDEN,), jnp.float32),
            pltpu.VMEM((C,), jnp.int32),
            pltpu.VMEM((C,), jnp.int32),
            pltpu.VMEM((C,), jnp.int32),
            pltpu.VMEM((2, HIDDEN), jnp.float32),
            pltpu.VMEM((HIDDEN,), jnp.float32),
            pltpu.VMEM((HIDDEN,), jnp.float32),
            pltpu.VMEM((C, HIDDEN), jnp.float32),
            pltpu.VMEM((C, HIDDEN), jnp.float32),
            pltpu.SemaphoreType.DMA,
        ],
    )
    return f(iw, ip, it, w, p, t, g, b)


def kernel(input_ids, position_ids, token_type_ids, word_embeddings,
           position_table, token_type_table, gamma, beta):
    iw = input_ids.reshape(N).astype(jnp.int32)
    ip = position_ids.reshape(N).astype(jnp.int32)
    it = token_type_ids.reshape(N).astype(jnp.int32)
    out = _run(iw, ip, it, word_embeddings, position_table.reshape(-1),
               token_type_table, gamma, beta)
    return out.reshape(B, L, HIDDEN)


# speedup vs baseline: 5.3534x; 1.3696x over previous
import jax
import jax.numpy as jnp
from jax import lax
from jax.experimental import pallas as pl
from jax.experimental.pallas import tpu as pltpu
from jax.experimental.pallas import tpu_sc as plsc

VOCAB = 100000
HIDDEN = 128
EPS = 1e-12
B, L = 1024, 200
N = B * L
NC, NS = 2, 16
NW = NC * NS
PER_W = N // NW
C = 128
G = PER_W // C
NV = HIDDEN // 16
POS_ROWS = 200


def _body(iw_hbm, ci_hbm, w_hbm, p_hbm, t_hbm, g_hbm, b_hbm,
          out_hbm, comb_v, idxw_v, ci_v, ttv, gamv, betv,
          rows_v, out_v, gsem0, gsem1, osem0, osem1):
    wid = lax.axis_index("s") * NC + lax.axis_index("c")
    base = wid * PER_W

    pw = POS_ROWS * HIDDEN
    pltpu.sync_copy(p_hbm.at[pl.ds(0, pw)], comb_v.at[pl.ds(0, pw)])
    pltpu.sync_copy(p_hbm.at[pl.ds(0, pw)], comb_v.at[pl.ds(pw, pw)])
    pltpu.sync_copy(t_hbm, ttv)
    pltpu.sync_copy(g_hbm, gamv)
    pltpu.sync_copy(b_hbm, betv)
    pltpu.sync_copy(iw_hbm.at[pl.ds(base, PER_W)], idxw_v)
    pltpu.sync_copy(ci_hbm.at[pl.ds(base, PER_W)], ci_v)

    tt0 = [ttv[0, pl.ds(16 * i, 16)] for i in range(NV)]
    tt1 = [ttv[1, pl.ds(16 * i, 16)] for i in range(NV)]

    @pl.loop(0, POS_ROWS)
    def _comb(r):
        for i in range(NV):
            sl0 = pl.ds(r * HIDDEN + 16 * i, 16)
            sl1 = pl.ds(pw + r * HIDDEN + 16 * i, 16)
            comb_v[sl0] = comb_v[sl0] + tt0[i]
            comb_v[sl1] = comb_v[sl1] + tt1[i]

    gam = [gamv[pl.ds(16 * i, 16)] for i in range(NV)]
    bet = [betv[pl.ds(16 * i, 16)] for i in range(NV)]
    lane = lax.iota(jnp.int32, 16)
    inv_h = jnp.float32(1.0 / HIDDEN)
    gsem = (gsem0, gsem1)
    osem = (osem0, osem1)

    def start_gather(sl, ch):
        pltpu.async_copy(w_hbm.at[idxw_v.at[pl.ds(ch * C, C)]],
                         rows_v.at[sl], gsem[sl])

    def wait_gather(sl):
        pltpu.make_async_copy(w_hbm.at[idxw_v.at[pl.ds(0, C)]],
                              rows_v.at[sl], gsem[sl]).wait()

    def start_out(sl, ch):
        pltpu.async_copy(out_v.at[sl], out_hbm.at[pl.ds(base + ch * C, C)],
                         osem[sl])

    def wait_out(sl):
        pltpu.make_async_copy(out_v.at[sl], out_hbm.at[pl.ds(0, C)],
                              osem[sl]).wait()

    def compute(sl, ch):
        rows = rows_v.at[sl]
        ov = out_v.at[sl]

        @pl.loop(0, C // 16)
        def _tokgrp(tg):
            ci_vec = ci_v[pl.ds(ch * C + 16 * tg, 16)] * HIDDEN
            for j in range(16):
                t = 16 * tg + j
                ci = ci_vec[j] + lane
                xs = []
                s = None
                q = None
                for i in range(NV):
                    w = rows[t, pl.ds(16 * i, 16)]
                    cvec = plsc.load_gather(comb_v, [ci + 16 * i])
                    x = w + cvec
                    xs.append(x)
                    s = x if s is None else s + x
                    q = x * x if q is None else q + x * x
                mean = jnp.sum(s) * inv_h
                var = jnp.maximum(jnp.sum(q) * inv_h - mean * mean, 0.0)
                v = var + jnp.float32(EPS)
                bits = lax.bitcast_convert_type(v, jnp.int32)
                y = lax.bitcast_convert_type(
                    jnp.int32(0x5F3759DF) - (bits >> 1), jnp.float32)
                half_v = 0.5 * v
                for _ in range(3):
                    y = y * (1.5 - half_v * y * y)
                ms = mean * y
                for i in range(NV):
                    ov[t, pl.ds(16 * i, 16)] = \
                        (xs[i] * y - ms) * gam[i] + bet[i]

    start_gather(0, 0)

    @pl.loop(0, G // 2)
    def _piter(k):
        c0 = 2 * k
        start_gather(1, c0 + 1)
        wait_gather(0)

        @pl.when(k > 0)
        def _():
            wait_out(0)

        compute(0, c0)
        start_out(0, c0)

        @pl.when(k < G // 2 - 1)
        def _():
            start_gather(0, c0 + 2)

        wait_gather(1)

        @pl.when(k > 0)
        def _():
            wait_out(1)

        compute(1, c0 + 1)
        start_out(1, c0 + 1)

    wait_out(0)
    wait_out(1)


@jax.jit
def _run(iw, ci, w, p, t, g, b):
    mesh = plsc.VectorSubcoreMesh(core_axis_name="c", subcore_axis_name="s",
                                  num_cores=NC, num_subcores=NS)
    f = pl.kernel(
        _body,
        out_type=jax.ShapeDtypeStruct((N, HIDDEN), jnp.float32),
        mesh=mesh,
        compiler_params=pltpu.CompilerParams(needs_layout_passes=False),
        scratch_types=[
            pltpu.VMEM((2 * POS_ROWS * HIDDEN,), jnp.float32),
            pltpu.VMEM((PER_W,), jnp.int32),
            pltpu.VMEM((PER_W,), jnp.int32),
            pltpu.VMEM((2, HIDDEN), jnp.float32),
            pltpu.VMEM((HIDDEN,), jnp.float32),
            pltpu.VMEM((HIDDEN,), jnp.float32),
            pltpu.VMEM((2, C, HIDDEN), jnp.float32),
            pltpu.VMEM((2, C, HIDDEN), jnp.float32),
            pltpu.SemaphoreType.DMA,
            pltpu.SemaphoreType.DMA,
            pltpu.SemaphoreType.DMA,
            pltpu.SemaphoreType.DMA,
        ],
    )
    return f(iw, ci, w, p, t, g, b)


def kernel(input_ids, position_ids, token_type_ids, word_embeddings,
           position_table, token_type_table, gamma, beta):
    iw = input_ids.reshape(N).astype(jnp.int32)
    ci = (position_ids.reshape(N).astype(jnp.int32)
          + token_type_ids.reshape(N).astype(jnp.int32) * POS_ROWS)
    out = _run(iw, ci, word_embeddings, position_table.reshape(-1),
               token_type_table, gamma, beta)
    return out.reshape(B, L, HIDDEN)
